# baseline (speedup 1.0000x reference)
import jax
import jax.numpy as jnp
from jax.experimental import pallas as pl
from jax.experimental.pallas import tpu as pltpu

_BLOCK = 512


def _body(ei_ref, rw_ref, se_ref, h_ref, o_ref):
    i = pl.program_id(0)
    ei = ei_ref[0]
    rw = rw_ref[pl.ds(i * _BLOCK, _BLOCK), :]
    se = se_ref[pl.ds(i * _BLOCK, _BLOCK), :]
    w = jnp.sum(jnp.where(se == ei, rw, 0.0), axis=-1, keepdims=True)
    o_ref[...] = h_ref[...] * w


def kernel(routing_weights, selected_experts, hidden_state, expert_idx):
    n, k = routing_weights.shape
    d = hidden_state.shape[1]
    ei = jnp.asarray(expert_idx, jnp.int32).reshape((1,))
    se = selected_experts.astype(jnp.int32)
    grid = (n // _BLOCK,)
    return pl.pallas_call(
        _body,
        grid=grid,
        in_specs=[
            pl.BlockSpec(memory_space=pltpu.SMEM),
            pl.BlockSpec((n, k), lambda i: (0, 0)),
            pl.BlockSpec((n, k), lambda i: (0, 0)),
            pl.BlockSpec((_BLOCK, d), lambda i: (i, 0)),
        ],
        out_specs=pl.BlockSpec((_BLOCK, d), lambda i: (i, 0)),
        out_shape=jax.ShapeDtypeStruct((n, d), hidden_state.dtype),
        compiler_params=pltpu.CompilerParams(
            dimension_semantics=("arbitrary",)),
    )(ei, routing_weights, se, hidden_state)


# speedup vs baseline: 1.1294x; 1.1294x over previous
import jax
import jax.numpy as jnp
from jax.experimental import pallas as pl
from jax.experimental.pallas import tpu as pltpu

_BLOCK = 512


def _body(ei_ref, rw_ref, se_ref, h_ref, o_ref):
    ei = ei_ref[0]
    a = jnp.where(se_ref[...] == ei, rw_ref[...], 0.0)
    ones = jnp.ones((8, 128), jnp.float32)
    w = jax.lax.dot_general(a, ones, (((0,), (0,)), ((), ())),
                            preferred_element_type=jnp.float32)
    o_ref[...] = h_ref[...] * w[:, 0:1]


def kernel(routing_weights, selected_experts, hidden_state, expert_idx):
    n, k = routing_weights.shape
    d = hidden_state.shape[1]
    ei = jnp.asarray(expert_idx, jnp.int32).reshape((1,))
    rw_t = jnp.pad(routing_weights.T, ((0, 8 - k), (0, 0)))
    se_t = jnp.pad(selected_experts.astype(jnp.int32).T, ((0, 8 - k), (0, 0)))
    grid = (n // _BLOCK,)
    return pl.pallas_call(
        _body,
        grid=grid,
        in_specs=[
            pl.BlockSpec(memory_space=pltpu.SMEM),
            pl.BlockSpec((8, _BLOCK), lambda i: (0, i)),
            pl.BlockSpec((8, _BLOCK), lambda i: (0, i)),
            pl.BlockSpec((_BLOCK, d), lambda i: (i, 0)),
        ],
        out_specs=pl.BlockSpec((_BLOCK, d), lambda i: (i, 0)),
        out_shape=jax.ShapeDtypeStruct((n, d), hidden_state.dtype),
        compiler_params=pltpu.CompilerParams(
            dimension_semantics=("arbitrary",)),
    )(ei, rw_t, se_t, hidden_state)


# speedup vs baseline: 1.1351x; 1.0051x over previous
import jax
import jax.numpy as jnp
from jax.experimental import pallas as pl
from jax.experimental.pallas import tpu as pltpu

_BLOCK = 512


def _body(p_ref, h_ref, o_ref):
    ei = p_ref[4:5, :]
    a = jnp.where(p_ref[2:4, :] == ei, p_ref[0:2, :], 0.0)
    ones = jnp.ones((2, 128), jnp.float32)
    w = jax.lax.dot_general(a, ones, (((0,), (0,)), ((), ())),
                            preferred_element_type=jnp.float32)
    o_ref[...] = h_ref[...] * w[:, 0:1]


def kernel(routing_weights, selected_experts, hidden_state, expert_idx):
    n, k = routing_weights.shape
    d = hidden_state.shape[1]
    ei_row = jnp.broadcast_to(
        jnp.asarray(expert_idx, jnp.float32), (1, n))
    packed = jnp.concatenate(
        [routing_weights.T,
         selected_experts.astype(jnp.float32).T,
         ei_row,
         jnp.full((7 - 2 * k, n), -1.0, jnp.float32)], axis=0)
    grid = (n // _BLOCK,)
    return pl.pallas_call(
        _body,
        grid=grid,
        in_specs=[
            pl.BlockSpec((8, _BLOCK), lambda i: (0, i)),
            pl.BlockSpec((_BLOCK, d), lambda i: (i, 0)),
        ],
        out_specs=pl.BlockSpec((_BLOCK, d), lambda i: (i, 0)),
        out_shape=jax.ShapeDtypeStruct((n, d), hidden_state.dtype),
        compiler_params=pltpu.CompilerParams(
            dimension_semantics=("arbitrary",)),
    )(packed, hidden_state)
